# baseline (speedup 1.0000x reference)
import jax
import jax.numpy as jnp
from jax import lax
from jax.experimental import pallas as pl
from jax.experimental.pallas import tpu as pltpu
from jax.experimental.pallas import tpu_sc as plsc

B = 8
H = 224
W = 224
C = 192
P = B * H * W
PIX_PER_IMG = H * W
NC = 2
NS = 16
NW = NC * NS
PIX_PER_W = P // NW
CHUNK = 64
N_CHUNKS = PIX_PER_W // CHUNK
N_PAIRS = N_CHUNKS // 2
LANES = 16


def _bf16_round(v):
    u = lax.bitcast_convert_type(v, jnp.int32)
    r = (u >> 16) & 1
    u = (u + 32767 + r) & jnp.int32(-65536)
    return lax.bitcast_convert_type(u, jnp.float32)


def _rne_int(x):
    big = jnp.float32(2.0 ** 23)
    pos = (x + big) - big
    neg = (x - big) + big
    return jnp.where(x >= 0.0, pos, neg).astype(jnp.int32)


def _sc_body(img_hbm, tform_hbm, out_hbm, tform_v,
             idx0, idx1, w0, w1, ga0, gb0, ga1, gb1, o0, o1,
             gsem0, gsem1, osem0, osem1):
    wid = lax.axis_index("s") * NC + lax.axis_index("c")
    wbase = wid * PIX_PER_W
    b = wid // 4
    bbase = b * PIX_PER_IMG

    idx = (idx0, idx1)
    wv = (w0, w1)
    ga = (ga0, ga1)
    gb = (gb0, gb1)
    ov = (o0, o1)
    gsem = (gsem0, gsem1)
    osem = (osem0, osem1)

    pltpu.sync_copy(tform_hbm, tform_v)
    trow = tform_v[b, :]
    trow_b = _bf16_round(trow)
    m00 = trow_b[0]
    m01 = trow_b[1]
    m10 = trow_b[2]
    m11 = trow_b[3]
    v0 = trow[4]
    v1 = trow[5]

    def fire(ci, s):
        gbase = wbase + ci * CHUNK
        for t in range(CHUNK // LANES):
            g = gbase + t * LANES + lax.iota(jnp.int32, LANES)
            rel = g - bbase
            i_i = rel // W
            j_i = rel % W
            si = i_i.astype(jnp.float32) / 223.0
            sj = j_i.astype(jnp.float32) / 223.0
            xt = _bf16_round(si - (1.0 - si))
            yt = _bf16_round(sj - (1.0 - sj))
            xs = (m00 * xt + m01 * yt) + v0
            ys = (m10 * xt + m11 * yt) + v1
            x = (0.5 * (xs + 1.0)) * 223.0
            y = (0.5 * (ys + 1.0)) * 223.0

            x0i = _rne_int(x)
            y0i = _rne_int(y)

            x0 = jnp.clip(x0i, 0, H)
            x1 = jnp.clip(x0i + 1, 0, H)
            y0 = jnp.clip(y0i, 0, W)
            y1 = jnp.clip(y0i + 1, 0, W)

            ax0 = x1.astype(jnp.float32) - x
            ax1 = x - x0.astype(jnp.float32)
            ay0 = y1.astype(jnp.float32) - y
            ay1 = y - y0.astype(jnp.float32)

            zero = jnp.zeros((LANES,), jnp.float32)
            vx0 = x0 < H
            vx1 = x1 < H
            vy0 = y0 < W
            vy1 = y1 < W
            w00 = jnp.where(vx0 & vy0, ax0 * ay0, zero)
            w01 = jnp.where(vx0 & vy1, ax0 * ay1, zero)
            w10 = jnp.where(vx1 & vy0, ax1 * ay0, zero)
            w11 = jnp.where(vx1 & vy1, ax1 * ay1, zero)

            xg0 = jnp.minimum(x0, H - 1)
            xg1 = jnp.minimum(x1, H - 1)
            yg0 = jnp.minimum(y0, W - 1)
            yg1 = jnp.minimum(y1, W - 1)
            base0 = bbase + xg0 * W
            base1 = bbase + xg1 * W
            sl = pl.ds(t * LANES, LANES)
            sl2 = pl.ds(CHUNK + t * LANES, LANES)
            idx[s][0, sl] = base0 + yg0
            idx[s][0, sl2] = base0 + yg1
            idx[s][1, sl] = base1 + yg0
            idx[s][1, sl2] = base1 + yg1
            wv[s][0, sl] = w00
            wv[s][1, sl] = w01
            wv[s][2, sl] = w10
            wv[s][3, sl] = w11

        pltpu.async_copy(img_hbm.at[idx[s].at[0]], ga[s], gsem[s])
        pltpu.async_copy(img_hbm.at[idx[s].at[1]], gb[s], gsem[s])

    def wait_gathers(s):
        pltpu.make_async_copy(img_hbm.at[idx[s].at[0]], ga[s], gsem[s]).wait()
        pltpu.make_async_copy(img_hbm.at[idx[s].at[1]], gb[s], gsem[s]).wait()

    def drain_out(s):
        pltpu.make_async_copy(ov[s], out_hbm.at[pl.ds(0, CHUNK)], osem[s]).wait()

    def blend_and_out(ci, s):
        gav = ga[s]
        gbv = gb[s]
        wvs = wv[s]
        ovs = ov[s]

        def blend(p, carry):
            pidx = jnp.full((LANES,), p, jnp.int32)
            k0 = jnp.zeros((LANES,), jnp.int32)
            w0v = plsc.load_gather(wvs, [k0, pidx])
            w1v = plsc.load_gather(wvs, [k0 + 1, pidx])
            w2v = plsc.load_gather(wvs, [k0 + 2, pidx])
            w3v = plsc.load_gather(wvs, [k0 + 3, pidx])
            p1 = p + CHUNK
            for cg in range(C // LANES):
                sc = pl.ds(cg * LANES, LANES)
                ovs[p, sc] = (gav[p, sc] * w0v + gav[p1, sc] * w1v
                              + gbv[p, sc] * w2v + gbv[p1, sc] * w3v)
            return carry

        lax.fori_loop(0, CHUNK, blend, 0, unroll=4)
        pltpu.async_copy(ovs, out_hbm.at[pl.ds(wbase + ci * CHUNK, CHUNK)],
                         osem[s])

    fire(0, 0)

    def step(g2, carry):
        c = 2 * g2
        fire(c + 1, 1)
        wait_gathers(0)

        @pl.when(g2 > 0)
        def _():
            drain_out(0)

        blend_and_out(c, 0)

        @pl.when(g2 < N_PAIRS - 1)
        def _():
            fire(c + 2, 0)

        wait_gathers(1)

        @pl.when(g2 > 0)
        def _():
            drain_out(1)

        blend_and_out(c + 1, 1)
        return carry

    lax.fori_loop(0, N_PAIRS, step, 0, unroll=False)
    drain_out(0)
    drain_out(1)


@jax.jit
def _apply2dtform_sc(img_flat, tform):
    mesh = plsc.VectorSubcoreMesh(core_axis_name="c", subcore_axis_name="s",
                                  num_cores=NC, num_subcores=NS)
    kfn = pl.kernel(
        _sc_body,
        out_type=jax.ShapeDtypeStruct((P, C), jnp.float32),
        mesh=mesh,
        compiler_params=pltpu.CompilerParams(use_tc_tiling_on_sc=False,
                                             needs_layout_passes=False),
        scratch_types=[
            pltpu.VMEM((B, 16), jnp.float32),
            pltpu.VMEM((2, 2 * CHUNK), jnp.int32),
            pltpu.VMEM((2, 2 * CHUNK), jnp.int32),
            pltpu.VMEM((4, CHUNK), jnp.float32),
            pltpu.VMEM((4, CHUNK), jnp.float32),
            pltpu.VMEM((2 * CHUNK, C), jnp.float32),
            pltpu.VMEM((2 * CHUNK, C), jnp.float32),
            pltpu.VMEM((2 * CHUNK, C), jnp.float32),
            pltpu.VMEM((2 * CHUNK, C), jnp.float32),
            pltpu.VMEM((CHUNK, C), jnp.float32),
            pltpu.VMEM((CHUNK, C), jnp.float32),
            pltpu.SemaphoreType.DMA,
            pltpu.SemaphoreType.DMA,
            pltpu.SemaphoreType.DMA,
            pltpu.SemaphoreType.DMA,
        ],
    )
    return kfn(img_flat, tform)


def kernel(Img, Tform):
    img_flat = Img.reshape(P, C)
    tform_pad = jnp.pad(Tform, ((0, 0), (0, 10)))
    out = _apply2dtform_sc(img_flat, tform_pad)
    return out.reshape(B, H, W, C)
